# baseline (speedup 1.0000x reference)
import functools

import jax
import jax.numpy as jnp
from jax import lax
from jax.experimental import pallas as pl
from jax.experimental.pallas import tpu as pltpu
from jax.experimental.pallas import tpu_sc as plsc

_PAD = 128


def _sc_gather(tab128, ids):
    B = ids.shape[0]
    info = plsc.get_sparse_core_info()
    nc, ns = info.num_cores, info.num_subcores
    nw = nc * ns
    b_per_w = B // nw
    mesh = plsc.VectorSubcoreMesh(core_axis_name="c", subcore_axis_name="s")

    @functools.partial(
        pl.kernel,
        mesh=mesh,
        compiler_params=pltpu.CompilerParams(use_tc_tiling_on_sc=True, needs_layout_passes=False),
        out_type=jax.ShapeDtypeStruct((B, _PAD), jnp.float32),
        scratch_types=[
            pltpu.VMEM((b_per_w,), jnp.int32),
            pltpu.VMEM((b_per_w,), jnp.int32),
            pltpu.VMEM((4, b_per_w // 4, _PAD), jnp.float32),
            pltpu.SemaphoreType.DMA,
            pltpu.SemaphoreType.DMA,
            pltpu.SemaphoreType.DMA,
            pltpu.SemaphoreType.DMA,
            pltpu.SemaphoreType.DMA,
            pltpu.SemaphoreType.DMA,
            pltpu.SemaphoreType.DMA,
            pltpu.SemaphoreType.DMA,
        ],
    )
    def gather_k(tab, tid, out, idx_v, sel_v, bufs,
                 g0, g1, g2, g3, o0, o1, o2, o3):
        wid = lax.axis_index("s") * nc + lax.axis_index("c")
        base = wid * b_per_w
        pltpu.sync_copy(tid.at[pl.ds(base, b_per_w)], idx_v)
        g32 = jnp.int32(_GROUP)

        def split_ids(k, _):
            ti = idx_v[pl.ds(16 * k, 16)]
            sel_v[pl.ds(16 * k, 16)] = lax.div(ti, g32)
            idx_v[pl.ds(16 * k, 16)] = lax.rem(ti, g32)
            return 0

        lax.fori_loop(0, b_per_w // 16, split_ids, 0)

        lanes = jnp.arange(16, dtype=jnp.int32)
        lane120 = jnp.full((16,), 120, dtype=jnp.int32)
        quarter = b_per_w // 4
        gsems = [g0, g1, g2, g3]
        osems = [o0, o1, o2, o3]

        def write_sel(q, off):
            def body(g, _):
                rows = 16 * g + lanes
                sel = sel_v[pl.ds(off + 16 * g, 16)]
                plsc.store_scatter(bufs.at[q], [rows, lane120],
                                   sel.astype(jnp.float32))
                return 0
            lax.fori_loop(0, quarter // 16, body, 0)

        gcps = [pltpu.async_copy(
            tab.at[idx_v.at[pl.ds(q * quarter, quarter)]],
            bufs.at[q], gsems[q]) for q in range(4)]
        ocps = []
        for q in range(4):
            gcps[q].wait()
            write_sel(q, q * quarter)
            ocps.append(pltpu.async_copy(
                bufs.at[q], out.at[pl.ds(base + q * quarter, quarter)],
                osems[q]))
        for cp in ocps:
            cp.wait()

    return gather_k(tab128, ids)


_GROUP = 50176


def _prep_body(p0, p1, eyes, po):
    f32 = jnp.float32
    dims = (((0,), (0,)), ((), ()))
    po[...] = (jax.lax.dot_general(p0[...], eyes[0, 0], dims,
                                   preferred_element_type=f32)
               + jax.lax.dot_general(p1[...], eyes[0, 1], dims,
                                     preferred_element_type=f32))


def _prep(tabT, eyes):
    C = 25088
    nb = _GROUP // C
    grid = (nb,)

    def tblock(q):
        return lambda i: (0, q * nb + i)

    return pl.pallas_call(
        _prep_body,
        grid=grid,
        in_specs=(
            [pl.BlockSpec((tabT.shape[0], C), tblock(q)) for q in range(2)]
            + [pl.BlockSpec((1, 2) + eyes.shape[2:], lambda i: (0, 0, 0, 0))]
        ),
        out_specs=pl.BlockSpec((C, _PAD), lambda i: (i, 0)),
        out_shape=jax.ShapeDtypeStruct((_GROUP, _PAD), jnp.float32),
    )(tabT, tabT, eyes)


def _mlp_body(mv, us, gnT, vt, wg, bg, w1m, w1u, w1g, w1v, b1,
              w2, b2, w3, b3, w4, b4, out):
    f32 = jnp.float32
    emb = w1m.shape[0]
    dims = (((0,), (0,)), ((), ()))
    g = jax.lax.dot_general(gnT[...], wg[...], dims,
                            preferred_element_type=f32) + bg[...]
    g = jnp.maximum(g, 0.0)
    mvr = mv[...]
    usr = us[...]
    mve = jnp.where(mvr[:, 120:121] < 0.5, mvr[:, 0:emb], mvr[:, 64:64 + emb])
    use = jnp.where(usr[:, 120:121] < 0.5, usr[:, 0:emb], usr[:, 64:64 + emb])
    x = (jnp.dot(mve, w1m[...], preferred_element_type=f32)
         + jnp.dot(use, w1u[...], preferred_element_type=f32)
         + jnp.dot(g, w1g[...], preferred_element_type=f32)
         + jax.lax.dot_general(vt[...], w1v[...], dims,
                               preferred_element_type=f32)
         + b1[...])
    x = jnp.maximum(x, 0.0)
    x = jnp.maximum(jnp.dot(x, w2[...], preferred_element_type=f32) + b2[...], 0.0)
    x = jnp.maximum(jnp.dot(x, w3[...], preferred_element_type=f32) + b3[...], 0.0)
    dt = (((0,), (1,)), ((), ()))
    xt = jnp.maximum(jax.lax.dot_general(w4[...], x, dt,
                                         preferred_element_type=f32)
                     + b4[...], 0.0)
    m = jnp.max(xt, axis=0, keepdims=True)
    e = jnp.exp(xt - m)
    out[...] = e / jnp.sum(e, axis=0, keepdims=True)


def _mlp(movieE, userE, genreT, vote, Wg, bg, W1m, W1u, W1g,
         w1v, b1, W2, b2, W3, b3, W4, b4):
    B = movieE.shape[0]
    T = 4096
    grid = (B // T,)

    def btile(minor):
        return pl.BlockSpec((T, minor), lambda i: (i, 0))

    def full(a):
        return pl.BlockSpec(a.shape, lambda i: (0, 0))

    return pl.pallas_call(
        _mlp_body,
        grid=grid,
        in_specs=[
            btile(movieE.shape[1]),
            btile(userE.shape[1]),
            pl.BlockSpec((genreT.shape[0], T), lambda i: (0, i)),
            pl.BlockSpec((1, T), lambda i: (0, i)),
            full(Wg), full(bg), full(W1m), full(W1u), full(W1g),
            full(w1v), full(b1), full(W2), full(b2), full(W3), full(b3),
            full(W4), full(b4),
        ],
        out_specs=pl.BlockSpec((5, T), lambda i: (0, i)),
        out_shape=jax.ShapeDtypeStruct((5, B), jnp.float32),
    )(movieE, userE, genreT, vote, Wg, bg, W1m, W1u, W1g, w1v,
      b1, W2, b2, W3, b3, W4, b4)


def kernel(userId, movieId, genre, vote_average, release_date, movie_table,
           user_table, Wg, bg, W1, b1, W2, b2, W3, b3, W4, b4):
    B = userId.shape[0]
    emb = movie_table.shape[1]
    mids = movieId.reshape(B)
    uids = userId.reshape(B)
    eyes = jnp.stack([jnp.eye(emb, _PAD, k=64 * q, dtype=jnp.float32)
                      for q in range(2)])[None]
    mt128 = _prep(movie_table.T, eyes)
    movieE = _sc_gather(mt128, mids)
    ut128 = _prep(user_table.T, eyes)
    userE = _sc_gather(ut128, uids)
    genreT = genre.reshape(B, genre.shape[-1]).T
    W1m = W1[0:20]
    W1u = W1[20:40]
    W1g = W1[40:48]
    w1v = W1[48:49]
    return _mlp(movieE, userE, genreT, vote_average.T,
                Wg, bg.reshape(1, -1), W1m, W1u, W1g, w1v, b1.reshape(1, -1),
                W2, b2.reshape(1, -1), W3, b3.reshape(1, -1),
                W4, b4.reshape(-1, 1)).T


# speedup vs baseline: 1.0099x; 1.0099x over previous
import functools

import jax
import jax.numpy as jnp
from jax import lax
from jax.experimental import pallas as pl
from jax.experimental.pallas import tpu as pltpu
from jax.experimental.pallas import tpu_sc as plsc

_PAD = 128


def _sc_gather(tab128, ids):
    B = ids.shape[0]
    info = plsc.get_sparse_core_info()
    nc, ns = info.num_cores, info.num_subcores
    nw = nc * ns
    b_per_w = B // nw
    mesh = plsc.VectorSubcoreMesh(core_axis_name="c", subcore_axis_name="s")

    @functools.partial(
        pl.kernel,
        mesh=mesh,
        compiler_params=pltpu.CompilerParams(use_tc_tiling_on_sc=True, needs_layout_passes=False),
        out_type=jax.ShapeDtypeStruct((B, _PAD), jnp.float32),
        scratch_types=[
            pltpu.VMEM((b_per_w,), jnp.int32),
            pltpu.VMEM((b_per_w,), jnp.int32),
            pltpu.VMEM((4, b_per_w // 4, _PAD), jnp.float32),
            pltpu.SemaphoreType.DMA,
            pltpu.SemaphoreType.DMA,
            pltpu.SemaphoreType.DMA,
            pltpu.SemaphoreType.DMA,
            pltpu.SemaphoreType.DMA,
            pltpu.SemaphoreType.DMA,
            pltpu.SemaphoreType.DMA,
            pltpu.SemaphoreType.DMA,
        ],
    )
    def gather_k(tab, tid, out, idx_v, sel_v, bufs,
                 g0, g1, g2, g3, o0, o1, o2, o3):
        wid = lax.axis_index("s") * nc + lax.axis_index("c")
        base = wid * b_per_w
        pltpu.sync_copy(tid.at[pl.ds(base, b_per_w)], idx_v)
        g32 = jnp.int32(_GROUP)

        def split_ids(k, _):
            ti = idx_v[pl.ds(16 * k, 16)]
            sel_v[pl.ds(16 * k, 16)] = lax.div(ti, g32)
            idx_v[pl.ds(16 * k, 16)] = lax.rem(ti, g32)
            return 0

        lax.fori_loop(0, b_per_w // 16, split_ids, 0)

        lanes = jnp.arange(16, dtype=jnp.int32)
        lane120 = jnp.full((16,), 120, dtype=jnp.int32)
        quarter = b_per_w // 4
        gsems = [g0, g1, g2, g3]
        osems = [o0, o1, o2, o3]

        def write_sel(q, off):
            def body(g, _):
                rows = 16 * g + lanes
                sel = sel_v[pl.ds(off + 16 * g, 16)]
                plsc.store_scatter(bufs.at[q], [rows, lane120],
                                   sel.astype(jnp.float32))
                return 0
            lax.fori_loop(0, quarter // 16, body, 0)

        gcps = [pltpu.async_copy(
            tab.at[idx_v.at[pl.ds(q * quarter, quarter)]],
            bufs.at[q], gsems[q]) for q in range(4)]
        ocps = []
        for q in range(4):
            gcps[q].wait()
            write_sel(q, q * quarter)
            ocps.append(pltpu.async_copy(
                bufs.at[q], out.at[pl.ds(base + q * quarter, quarter)],
                osems[q]))
        for cp in ocps:
            cp.wait()

    return gather_k(tab128, ids)


_GROUP = 50176


def _prep_body(p0, p1, eyes, po):
    f32 = jnp.float32
    dims = (((0,), (0,)), ((), ()))
    po[...] = (jax.lax.dot_general(p0[...], eyes[0, 0], dims,
                                   preferred_element_type=f32)
               + jax.lax.dot_general(p1[...], eyes[0, 1], dims,
                                     preferred_element_type=f32))


def _prep(tabT, eyes):
    C = 12544
    nb = _GROUP // C
    grid = (nb,)

    def tblock(q):
        return lambda i: (0, q * nb + i)

    return pl.pallas_call(
        _prep_body,
        grid=grid,
        in_specs=(
            [pl.BlockSpec((tabT.shape[0], C), tblock(q)) for q in range(2)]
            + [pl.BlockSpec((1, 2) + eyes.shape[2:], lambda i: (0, 0, 0, 0))]
        ),
        out_specs=pl.BlockSpec((C, _PAD), lambda i: (i, 0)),
        out_shape=jax.ShapeDtypeStruct((_GROUP, _PAD), jnp.float32),
    )(tabT, tabT, eyes)


def _mlp_body(mv, us, gnT, vt, wg, bg, w1m, w1u, w1g, w1v, b1,
              w2, b2, w3, b3, w4, b4, out):
    f32 = jnp.float32
    emb = w1m.shape[0]
    dims = (((0,), (0,)), ((), ()))
    g = jax.lax.dot_general(gnT[...], wg[...], dims,
                            preferred_element_type=f32) + bg[...]
    g = jnp.maximum(g, 0.0)
    mvr = mv[...]
    usr = us[...]
    mve = jnp.where(mvr[:, 120:121] < 0.5, mvr[:, 0:emb], mvr[:, 64:64 + emb])
    use = jnp.where(usr[:, 120:121] < 0.5, usr[:, 0:emb], usr[:, 64:64 + emb])
    x = (jnp.dot(mve, w1m[...], preferred_element_type=f32)
         + jnp.dot(use, w1u[...], preferred_element_type=f32)
         + jnp.dot(g, w1g[...], preferred_element_type=f32)
         + jax.lax.dot_general(vt[...], w1v[...], dims,
                               preferred_element_type=f32)
         + b1[...])
    x = jnp.maximum(x, 0.0)
    x = jnp.maximum(jnp.dot(x, w2[...], preferred_element_type=f32) + b2[...], 0.0)
    x = jnp.maximum(jnp.dot(x, w3[...], preferred_element_type=f32) + b3[...], 0.0)
    dt = (((0,), (1,)), ((), ()))
    xt = jnp.maximum(jax.lax.dot_general(w4[...], x, dt,
                                         preferred_element_type=f32)
                     + b4[...], 0.0)
    m = jnp.max(xt, axis=0, keepdims=True)
    e = jnp.exp(xt - m)
    out[...] = e / jnp.sum(e, axis=0, keepdims=True)


def _mlp(movieE, userE, genreT, vote, Wg, bg, W1m, W1u, W1g,
         w1v, b1, W2, b2, W3, b3, W4, b4):
    B = movieE.shape[0]
    T = 8192
    grid = (B // T,)

    def btile(minor):
        return pl.BlockSpec((T, minor), lambda i: (i, 0))

    def full(a):
        return pl.BlockSpec(a.shape, lambda i: (0, 0))

    return pl.pallas_call(
        _mlp_body,
        grid=grid,
        in_specs=[
            btile(movieE.shape[1]),
            btile(userE.shape[1]),
            pl.BlockSpec((genreT.shape[0], T), lambda i: (0, i)),
            pl.BlockSpec((1, T), lambda i: (0, i)),
            full(Wg), full(bg), full(W1m), full(W1u), full(W1g),
            full(w1v), full(b1), full(W2), full(b2), full(W3), full(b3),
            full(W4), full(b4),
        ],
        out_specs=pl.BlockSpec((5, T), lambda i: (0, i)),
        out_shape=jax.ShapeDtypeStruct((5, B), jnp.float32),
    )(movieE, userE, genreT, vote, Wg, bg, W1m, W1u, W1g, w1v,
      b1, W2, b2, W3, b3, W4, b4)


def kernel(userId, movieId, genre, vote_average, release_date, movie_table,
           user_table, Wg, bg, W1, b1, W2, b2, W3, b3, W4, b4):
    B = userId.shape[0]
    emb = movie_table.shape[1]
    mids = movieId.reshape(B)
    uids = userId.reshape(B)
    eyes = jnp.stack([jnp.eye(emb, _PAD, k=64 * q, dtype=jnp.float32)
                      for q in range(2)])[None]
    mt128 = _prep(movie_table.T, eyes)
    movieE = _sc_gather(mt128, mids)
    ut128 = _prep(user_table.T, eyes)
    userE = _sc_gather(ut128, uids)
    genreT = genre.reshape(B, genre.shape[-1]).T
    W1m = W1[0:20]
    W1u = W1[20:40]
    W1g = W1[40:48]
    w1v = W1[48:49]
    return _mlp(movieE, userE, genreT, vote_average.T,
                Wg, bg.reshape(1, -1), W1m, W1u, W1g, w1v, b1.reshape(1, -1),
                W2, b2.reshape(1, -1), W3, b3.reshape(1, -1),
                W4, b4.reshape(-1, 1)).T


# speedup vs baseline: 1.0275x; 1.0173x over previous
import functools

import jax
import jax.numpy as jnp
from jax import lax
from jax.experimental import pallas as pl
from jax.experimental.pallas import tpu as pltpu
from jax.experimental.pallas import tpu_sc as plsc

_PAD = 128


def _sc_gather(tab128, ids):
    B = ids.shape[0]
    info = plsc.get_sparse_core_info()
    nc, ns = info.num_cores, info.num_subcores
    nw = nc * ns
    b_per_w = B // nw
    mesh = plsc.VectorSubcoreMesh(core_axis_name="c", subcore_axis_name="s")

    @functools.partial(
        pl.kernel,
        mesh=mesh,
        compiler_params=pltpu.CompilerParams(use_tc_tiling_on_sc=True, needs_layout_passes=False),
        out_type=jax.ShapeDtypeStruct((B, _PAD), jnp.float32),
        scratch_types=[
            pltpu.VMEM((b_per_w,), jnp.int32),
            pltpu.VMEM((b_per_w,), jnp.int32),
            pltpu.VMEM((4, b_per_w // 4, _PAD), jnp.float32),
            pltpu.SemaphoreType.DMA,
            pltpu.SemaphoreType.DMA,
            pltpu.SemaphoreType.DMA,
            pltpu.SemaphoreType.DMA,
            pltpu.SemaphoreType.DMA,
            pltpu.SemaphoreType.DMA,
            pltpu.SemaphoreType.DMA,
            pltpu.SemaphoreType.DMA,
        ],
    )
    def gather_k(tab, tid, out, idx_v, sel_v, bufs,
                 g0, g1, g2, g3, o0, o1, o2, o3):
        wid = lax.axis_index("s") * nc + lax.axis_index("c")
        base = wid * b_per_w
        pltpu.sync_copy(tid.at[pl.ds(base, b_per_w)], idx_v)
        g32 = jnp.int32(_GROUP)

        def split_ids(k, _):
            ti = idx_v[pl.ds(16 * k, 16)]
            sel_v[pl.ds(16 * k, 16)] = lax.div(ti, g32)
            idx_v[pl.ds(16 * k, 16)] = lax.rem(ti, g32)
            return 0

        lax.fori_loop(0, b_per_w // 16, split_ids, 0)

        lanes = jnp.arange(16, dtype=jnp.int32)
        lane120 = jnp.full((16,), 120, dtype=jnp.int32)
        quarter = b_per_w // 4
        gsems = [g0, g1, g2, g3]
        osems = [o0, o1, o2, o3]

        def write_sel(q, off):
            def body(g, _):
                rows = 16 * g + lanes
                sel = sel_v[pl.ds(off + 16 * g, 16)]
                plsc.store_scatter(bufs.at[q], [rows, lane120],
                                   sel.astype(jnp.float32))
                return 0
            lax.fori_loop(0, quarter // 16, body, 0)

        gcps = [pltpu.async_copy(
            tab.at[idx_v.at[pl.ds(q * quarter, quarter)]],
            bufs.at[q], gsems[q]) for q in range(4)]
        ocps = []
        for q in range(4):
            gcps[q].wait()
            write_sel(q, q * quarter)
            ocps.append(pltpu.async_copy(
                bufs.at[q], out.at[pl.ds(base + q * quarter, quarter)],
                osems[q]))
        for cp in ocps:
            cp.wait()

    return gather_k(tab128, ids)


_GROUP = 50176


def _prep_body(p0, p1, eyes, po):
    f32 = jnp.float32
    dims = (((0,), (0,)), ((), ()))
    po[...] = (jax.lax.dot_general(p0[...], eyes[0, 0], dims,
                                   preferred_element_type=f32)
               + jax.lax.dot_general(p1[...], eyes[0, 1], dims,
                                     preferred_element_type=f32))


def _prep(tabT, eyes):
    C = 12544
    nb = _GROUP // C
    grid = (nb,)

    def tblock(q):
        return lambda i: (0, q * nb + i)

    return pl.pallas_call(
        _prep_body,
        grid=grid,
        in_specs=(
            [pl.BlockSpec((tabT.shape[0], C), tblock(q)) for q in range(2)]
            + [pl.BlockSpec((1, 2) + eyes.shape[2:], lambda i: (0, 0, 0, 0))]
        ),
        out_specs=pl.BlockSpec((C, _PAD), lambda i: (i, 0)),
        out_shape=jax.ShapeDtypeStruct((_GROUP, _PAD), jnp.float32),
    )(tabT, tabT, eyes)


def _mlp_body(mv, us, gnT, vt, wg, bg, w1m, w1u, w1g, w1v, b1,
              w2, b2, w3, b3, w4, b4, out):
    f32 = jnp.float32
    emb = w1m.shape[0]
    dims = (((0,), (0,)), ((), ()))
    g = jax.lax.dot_general(gnT[...], wg[...], dims,
                            preferred_element_type=f32) + bg[...]
    g = jnp.maximum(g, 0.0)
    mvr = mv[...]
    usr = us[...]
    mve = jnp.where(mvr[:, 120:121] < 0.5, mvr[:, 0:emb], mvr[:, 64:64 + emb])
    use = jnp.where(usr[:, 120:121] < 0.5, usr[:, 0:emb], usr[:, 64:64 + emb])
    x = (jnp.dot(mve, w1m[...], preferred_element_type=f32)
         + jnp.dot(use, w1u[...], preferred_element_type=f32)
         + jnp.dot(g, w1g[...], preferred_element_type=f32)
         + jax.lax.dot_general(vt[...], w1v[...], dims,
                               preferred_element_type=f32)
         + b1[...])
    x = jnp.maximum(x, 0.0)
    x = jnp.maximum(jnp.dot(x, w2[...], preferred_element_type=f32) + b2[...], 0.0)
    x = jnp.maximum(jnp.dot(x, w3[...], preferred_element_type=f32) + b3[...], 0.0)
    dt = (((0,), (1,)), ((), ()))
    xt = jnp.maximum(jax.lax.dot_general(w4[...], x, dt,
                                         preferred_element_type=f32)
                     + b4[...], 0.0)
    m = jnp.max(xt, axis=0, keepdims=True)
    e = jnp.exp(xt - m)
    out[...] = e / jnp.sum(e, axis=0, keepdims=True)


def _mlp(movieE, userE, genreT, vote, Wg, bg, W1m, W1u, W1g,
         w1v, b1, W2, b2, W3, b3, W4, b4):
    B = movieE.shape[0]
    T = 4096
    grid = (B // T,)

    def btile(minor):
        return pl.BlockSpec((T, minor), lambda i: (i, 0))

    def full(a):
        return pl.BlockSpec(a.shape, lambda i: (0, 0))

    return pl.pallas_call(
        _mlp_body,
        grid=grid,
        in_specs=[
            btile(movieE.shape[1]),
            btile(userE.shape[1]),
            pl.BlockSpec((genreT.shape[0], T), lambda i: (0, i)),
            pl.BlockSpec((1, T), lambda i: (0, i)),
            full(Wg), full(bg), full(W1m), full(W1u), full(W1g),
            full(w1v), full(b1), full(W2), full(b2), full(W3), full(b3),
            full(W4), full(b4),
        ],
        out_specs=pl.BlockSpec((5, T), lambda i: (0, i)),
        out_shape=jax.ShapeDtypeStruct((5, B), jnp.float32),
    )(movieE, userE, genreT, vote, Wg, bg, W1m, W1u, W1g, w1v,
      b1, W2, b2, W3, b3, W4, b4)


def kernel(userId, movieId, genre, vote_average, release_date, movie_table,
           user_table, Wg, bg, W1, b1, W2, b2, W3, b3, W4, b4):
    B = userId.shape[0]
    emb = movie_table.shape[1]
    mids = movieId.reshape(B)
    uids = userId.reshape(B)
    eyes = jnp.stack([jnp.eye(emb, _PAD, k=64 * q, dtype=jnp.float32)
                      for q in range(2)])[None]
    mt128 = _prep(movie_table.T, eyes)
    movieE = _sc_gather(mt128, mids)
    ut128 = _prep(user_table.T, eyes)
    userE = _sc_gather(ut128, uids)
    genreT = genre.reshape(B, genre.shape[-1]).T
    W1m = W1[0:20]
    W1u = W1[20:40]
    W1g = W1[40:48]
    w1v = W1[48:49]
    return _mlp(movieE, userE, genreT, vote_average.T,
                Wg, bg.reshape(1, -1), W1m, W1u, W1g, w1v, b1.reshape(1, -1),
                W2, b2.reshape(1, -1), W3, b3.reshape(1, -1),
                W4, b4.reshape(-1, 1)).T
